# baseline (speedup 1.0000x reference)
import functools

import jax
import jax.numpy as jnp
from jax import lax
from jax.experimental import pallas as pl
from jax.experimental.pallas import tpu as pltpu
from jax.experimental.pallas import tpu_sc as plsc

NUM_CLASSES = 80
IOU_THRESH = 0.5

M_GT = 500
M_PAD = 512
N_PROP = 20000
NW = 32
PPW = 640
N_PAD = NW * PPW
L = 16

BPX = 7
NBINS = BPX * BPX
ROWLEN = 512
NBV = 4


def _body(px1h, py1h, px2h, py2h, gx1h, gy1h, gx2h, gy2h, gch,
          vals_h, idxs_h, cls_h,
          px1, py1, px2, py2, pbin,
          gx1, gy1, gx2, gy2, gc, gcomb, binfo,
          binlist, lens, sortidx, pcnt,
          ov, oi, oc):
    nc = plsc.get_sparse_core_info().num_cores
    wid = lax.axis_index("s") * nc + lax.axis_index("c")
    base = wid * PPW

    pltpu.sync_copy(px1h.at[pl.ds(base, PPW)], px1)
    pltpu.sync_copy(py1h.at[pl.ds(base, PPW)], py1)
    pltpu.sync_copy(px2h.at[pl.ds(base, PPW)], px2)
    pltpu.sync_copy(py2h.at[pl.ds(base, PPW)], py2)
    pltpu.sync_copy(gx1h, gx1)
    pltpu.sync_copy(gy1h, gy1)
    pltpu.sync_copy(gx2h, gx2)
    pltpu.sync_copy(gy2h, gy2)
    pltpu.sync_copy(gch, gc)

    iota = lax.iota(jnp.int32, L)
    zero = jnp.zeros((L,), jnp.int32)

    def gt_prep(m, _):
        s = pl.ds(m * L, L)
        gcomb[s] = ((iota + m * L) << 7) | gc[s]
        return 0

    lax.fori_loop(0, M_PAD // L, gt_prep, 0)

    def pbin_prep(i, _):
        s = pl.ds(i * L, L)
        ix = jnp.clip(px1[s].astype(jnp.int32) >> 7, 0, BPX - 1)
        iy = jnp.clip(py1[s].astype(jnp.int32) >> 7, 0, BPX - 1)
        pbin[s] = iy * BPX + ix
        return 0

    lax.fori_loop(0, PPW // L, pbin_prep, 0)

    for i in range(NBV):
        lens[pl.ds(i * L, L)] = zero

    def binfo_prep(c, _):
        s = pl.ds(c * L, L)
        bx1 = gx1[s]
        by1 = gy1[s]
        bx2 = gx2[s]
        by2 = gy2[s]
        kx = bx2.astype(jnp.int32) >> 7
        ky = by2.astype(jnp.int32) >> 7
        ixlo = jnp.maximum((bx1.astype(jnp.int32) >> 7) - 1, 0)
        iylo = jnp.maximum((by1.astype(jnp.int32) >> 7) - 1, 0)
        ixhi = jnp.minimum(
            kx - (bx2 <= (kx << 7).astype(jnp.float32)).astype(jnp.int32),
            BPX - 1)
        iyhi = jnp.minimum(
            ky - (by2 <= (ky << 7).astype(jnp.float32)).astype(jnp.int32),
            BPX - 1)
        binfo[s] = ixlo | (ixhi << 4) | (iylo << 8) | (iyhi << 12)
        return 0

    lax.fori_loop(0, M_PAD // L, binfo_prep, 0)

    ixc = [(iota + v * L) - ((iota + v * L) // BPX) * BPX for v in range(NBV)]
    iyc = [(iota + v * L) // BPX for v in range(NBV)]
    grbs = [(iota + v * L) * ROWLEN for v in range(NBV)]

    def gt_insert(m, cur):
        c = plsc.load_gather(binfo, [zero + m])
        cxlo = c & 15
        cxhi = (c >> 4) & 15
        cylo = (c >> 8) & 15
        cyhi = c >> 12
        ncur = []
        for v in range(NBV):
            valid = ((ixc[v] >= cxlo) & (ixc[v] <= cxhi)
                     & (iyc[v] >= cylo) & (iyc[v] <= cyhi))
            plsc.store_scatter(binlist, [grbs[v] + cur[v]], zero + m,
                               mask=valid)
            ncur.append(cur[v] + valid.astype(jnp.int32))
        return tuple(ncur)

    gcur = plsc.parallel_loop(0, M_GT, unroll=2,
                              carry=tuple(zero for _ in range(NBV)))(gt_insert)
    for v in range(NBV):
        lens[pl.ds(v * L, L)] = gcur[v]

    bcs = [iota + v * L for v in range(NBV)]
    rbs = [bcs[v] * PPW for v in range(NBV)]

    def ins_step(p, cur):
        bp = plsc.load_gather(pbin, [zero + p])
        ncur = []
        for v in range(NBV):
            e = bcs[v] == bp
            plsc.store_scatter(sortidx, [rbs[v] + cur[v]], zero + p, mask=e)
            ncur.append(cur[v] + e.astype(jnp.int32))
        return tuple(ncur)

    cur = plsc.parallel_loop(0, PPW, unroll=2,
                             carry=tuple(zero for _ in range(NBV)))(ins_step)
    for v in range(NBV):
        pcnt[pl.ds(v * L, L)] = cur[v]

    def bin_step(b, _):
        bsp = zero + b
        lenb = jnp.max(plsc.load_gather(lens, [bsp]))
        cntb = jnp.max(plsc.load_gather(pcnt, [bsp]))
        lbase = b * ROWLEN

        def pv_step(pv, _):
            so = b * PPW + pv * L
            oidx = sortidx[pl.ds(so, L)]
            act = (iota + pv * L) < cntb
            p1 = plsc.load_gather(px1, [oidx], mask=act)
            q1 = plsc.load_gather(py1, [oidx], mask=act)
            p2 = plsc.load_gather(px2, [oidx], mask=act)
            q2 = plsc.load_gather(py2, [oidx], mask=act)
            pa = (p2 - p1) * (q2 - q1)

            def k_step(k, carry):
                best, bcomb = carry
                gi = plsc.load_gather(binlist, [zero + (lbase + k)])
                bx1 = plsc.load_gather(gx1, [gi])
                by1 = plsc.load_gather(gy1, [gi])
                bx2 = plsc.load_gather(gx2, [gi])
                by2 = plsc.load_gather(gy2, [gi])
                combv = plsc.load_gather(gcomb, [gi])
                barea = (bx2 - bx1) * (by2 - by1)
                ltx = jnp.maximum(bx1, p1)
                lty = jnp.maximum(by1, q1)
                rbx = jnp.minimum(bx2, p2)
                rby = jnp.minimum(by2, q2)
                wx = jnp.maximum(rbx - ltx, 0.0)
                wy = jnp.maximum(rby - lty, 0.0)
                inter = wx * wy
                union = (barea + pa) - inter
                iou = inter / union
                upd = iou > best
                return (jnp.where(upd, iou, best),
                        jnp.where(upd, combv, bcomb))

            init = (jnp.zeros((L,), jnp.float32), jnp.zeros((L,), jnp.int32))
            best, bcomb = plsc.parallel_loop(0, lenb, unroll=8,
                                             carry=init)(k_step)

            fg = best >= IOU_THRESH
            plsc.store_scatter(ov, [oidx], best, mask=act)
            plsc.store_scatter(oi, [oidx], bcomb >> 7, mask=act)
            plsc.store_scatter(oc, [oidx],
                               jnp.where(fg, bcomb & 127, NUM_CLASSES),
                               mask=act)
            return 0

        lax.fori_loop(0, (cntb + L - 1) >> 4, pv_step, 0)
        return 0

    lax.fori_loop(0, NBINS, bin_step, 0)

    pltpu.sync_copy(ov, vals_h.at[pl.ds(base, PPW)])
    pltpu.sync_copy(oi, idxs_h.at[pl.ds(base, PPW)])
    pltpu.sync_copy(oc, cls_h.at[pl.ds(base, PPW)])


@jax.jit
def kernel(proposal_boxes, gt_boxes, gt_classes):
    pb = jnp.zeros((N_PAD, 4), jnp.float32).at[:N_PROP].set(proposal_boxes)
    gt = jnp.zeros((M_PAD, 4), jnp.float32).at[:M_GT].set(gt_boxes)
    gc = jnp.zeros((M_PAD,), jnp.int32).at[:M_GT].set(
        gt_classes.astype(jnp.int32))

    mesh = plsc.VectorSubcoreMesh(core_axis_name="c", subcore_axis_name="s")
    k = functools.partial(
        pl.kernel,
        mesh=mesh,
        compiler_params=pltpu.CompilerParams(needs_layout_passes=False),
        out_type=[
            jax.ShapeDtypeStruct((N_PAD,), jnp.float32),
            jax.ShapeDtypeStruct((N_PAD,), jnp.int32),
            jax.ShapeDtypeStruct((N_PAD,), jnp.int32),
        ],
        scratch_types=[
            pltpu.VMEM((PPW,), jnp.float32),
            pltpu.VMEM((PPW,), jnp.float32),
            pltpu.VMEM((PPW,), jnp.float32),
            pltpu.VMEM((PPW,), jnp.float32),
            pltpu.VMEM((PPW,), jnp.int32),
            pltpu.VMEM((M_PAD,), jnp.float32),
            pltpu.VMEM((M_PAD,), jnp.float32),
            pltpu.VMEM((M_PAD,), jnp.float32),
            pltpu.VMEM((M_PAD,), jnp.float32),
            pltpu.VMEM((M_PAD,), jnp.int32),
            pltpu.VMEM((M_PAD,), jnp.int32),
            pltpu.VMEM((M_PAD,), jnp.int32),
            pltpu.VMEM((NBINS * ROWLEN,), jnp.int32),
            pltpu.VMEM((NBV * L,), jnp.int32),
            pltpu.VMEM((NBINS * PPW,), jnp.int32),
            pltpu.VMEM((NBV * L,), jnp.int32),
            pltpu.VMEM((PPW,), jnp.float32),
            pltpu.VMEM((PPW,), jnp.int32),
            pltpu.VMEM((PPW,), jnp.int32),
        ],
    )(_body)

    vals, idxs, cls = k(
        pb[:, 0], pb[:, 1], pb[:, 2], pb[:, 3],
        gt[:, 0], gt[:, 1], gt[:, 2], gt[:, 3], gc,
    )
    return vals[:N_PROP], idxs[:N_PROP], cls[:N_PROP]


# speedup vs baseline: 1.0875x; 1.0875x over previous
import functools

import jax
import jax.numpy as jnp
from jax import lax
from jax.experimental import pallas as pl
from jax.experimental.pallas import tpu as pltpu
from jax.experimental.pallas import tpu_sc as plsc

NUM_CLASSES = 80
IOU_THRESH = 0.5

M_GT = 500
M_PAD = 512
N_PROP = 20000
NW = 32
PPW = 640
N_PAD = NW * PPW
L = 16

BPX = 7
NBINS = BPX * BPX
ROWLEN = 512
NBV = 4


def _body(px1h, py1h, px2h, py2h, gx1h, gy1h, gx2h, gy2h, gch,
          vals_h, idxs_h, cls_h,
          px1, py1, px2, py2, pbin,
          gx1, gy1, gx2, gy2, gc, gcomb, binfo,
          binlist, lens, sortidx, pcnt,
          ov, oi, oc):
    nc = plsc.get_sparse_core_info().num_cores
    wid = lax.axis_index("s") * nc + lax.axis_index("c")
    base = wid * PPW

    pltpu.sync_copy(px1h.at[pl.ds(base, PPW)], px1)
    pltpu.sync_copy(py1h.at[pl.ds(base, PPW)], py1)
    pltpu.sync_copy(px2h.at[pl.ds(base, PPW)], px2)
    pltpu.sync_copy(py2h.at[pl.ds(base, PPW)], py2)
    pltpu.sync_copy(gx1h, gx1)
    pltpu.sync_copy(gy1h, gy1)
    pltpu.sync_copy(gx2h, gx2)
    pltpu.sync_copy(gy2h, gy2)
    pltpu.sync_copy(gch, gc)

    iota = lax.iota(jnp.int32, L)
    zero = jnp.zeros((L,), jnp.int32)

    def gt_prep(m, _):
        s = pl.ds(m * L, L)
        gcomb[s] = ((iota + m * L) << 7) | gc[s]
        return 0

    lax.fori_loop(0, M_PAD // L, gt_prep, 0)

    def pbin_prep(i, _):
        s = pl.ds(i * L, L)
        ix = jnp.clip(px1[s].astype(jnp.int32) >> 7, 0, BPX - 1)
        iy = jnp.clip(py1[s].astype(jnp.int32) >> 7, 0, BPX - 1)
        pbin[s] = iy * BPX + ix
        return 0

    lax.fori_loop(0, PPW // L, pbin_prep, 0)

    for i in range(NBV):
        lens[pl.ds(i * L, L)] = zero

    def binfo_prep(c, _):
        s = pl.ds(c * L, L)
        bx1 = gx1[s]
        by1 = gy1[s]
        bx2 = gx2[s]
        by2 = gy2[s]
        kx = bx2.astype(jnp.int32) >> 7
        ky = by2.astype(jnp.int32) >> 7
        ixlo = jnp.maximum((bx1.astype(jnp.int32) >> 7) - 1, 0)
        iylo = jnp.maximum((by1.astype(jnp.int32) >> 7) - 1, 0)
        ixhi = jnp.minimum(
            kx - (bx2 <= (kx << 7).astype(jnp.float32)).astype(jnp.int32),
            BPX - 1)
        iyhi = jnp.minimum(
            ky - (by2 <= (ky << 7).astype(jnp.float32)).astype(jnp.int32),
            BPX - 1)
        binfo[s] = ixlo | (ixhi << 4) | (iylo << 8) | (iyhi << 12)
        return 0

    lax.fori_loop(0, M_PAD // L, binfo_prep, 0)

    ixc = [(iota + v * L) - ((iota + v * L) // BPX) * BPX for v in range(NBV)]
    iyc = [(iota + v * L) // BPX for v in range(NBV)]
    grbs = [(iota + v * L) * ROWLEN for v in range(NBV)]

    def gt_insert(m, cur):
        c = plsc.load_gather(binfo, [zero + m])
        cxlo = c & 15
        cxhi = (c >> 4) & 15
        cylo = (c >> 8) & 15
        cyhi = c >> 12
        ncur = []
        for v in range(NBV):
            valid = ((ixc[v] >= cxlo) & (ixc[v] <= cxhi)
                     & (iyc[v] >= cylo) & (iyc[v] <= cyhi))
            plsc.store_scatter(binlist, [grbs[v] + cur[v]], zero + m,
                               mask=valid)
            ncur.append(cur[v] + valid.astype(jnp.int32))
        return tuple(ncur)

    gcur = plsc.parallel_loop(0, M_GT, unroll=2,
                              carry=tuple(zero for _ in range(NBV)))(gt_insert)
    for v in range(NBV):
        lens[pl.ds(v * L, L)] = gcur[v]

    bcs = [iota + v * L for v in range(NBV)]
    rbs = [bcs[v] * PPW for v in range(NBV)]

    def ins_step(p, cur):
        bp = plsc.load_gather(pbin, [zero + p])
        ncur = []
        for v in range(NBV):
            e = bcs[v] == bp
            plsc.store_scatter(sortidx, [rbs[v] + cur[v]], zero + p, mask=e)
            ncur.append(cur[v] + e.astype(jnp.int32))
        return tuple(ncur)

    cur = plsc.parallel_loop(0, PPW, unroll=2,
                             carry=tuple(zero for _ in range(NBV)))(ins_step)
    for v in range(NBV):
        pcnt[pl.ds(v * L, L)] = cur[v]

    def bin_step(b, _):
        bsp = zero + b
        lenb = jnp.max(plsc.load_gather(lens, [bsp]))
        cntb = jnp.max(plsc.load_gather(pcnt, [bsp]))
        lbase = b * ROWLEN

        def pv_step(pv, _):
            so = b * PPW + pv * L
            oidx = sortidx[pl.ds(so, L)]
            act = (iota + pv * L) < cntb
            p1 = plsc.load_gather(px1, [oidx], mask=act)
            q1 = plsc.load_gather(py1, [oidx], mask=act)
            p2 = plsc.load_gather(px2, [oidx], mask=act)
            q2 = plsc.load_gather(py2, [oidx], mask=act)
            pa = (p2 - p1) * (q2 - q1)

            def k_step(k, carry):
                best, bcomb = carry
                gi = plsc.load_gather(binlist, [zero + (lbase + k)])
                bx1 = plsc.load_gather(gx1, [gi])
                by1 = plsc.load_gather(gy1, [gi])
                bx2 = plsc.load_gather(gx2, [gi])
                by2 = plsc.load_gather(gy2, [gi])
                combv = plsc.load_gather(gcomb, [gi])
                barea = (bx2 - bx1) * (by2 - by1)
                ltx = jnp.maximum(bx1, p1)
                lty = jnp.maximum(by1, q1)
                rbx = jnp.minimum(bx2, p2)
                rby = jnp.minimum(by2, q2)
                wx = jnp.maximum(rbx - ltx, 0.0)
                wy = jnp.maximum(rby - lty, 0.0)
                inter = wx * wy
                union = (barea + pa) - inter
                iou = inter / union
                upd = iou > best
                return (jnp.where(upd, iou, best),
                        jnp.where(upd, combv, bcomb))

            init = (jnp.zeros((L,), jnp.float32), jnp.zeros((L,), jnp.int32))
            best, bcomb = plsc.parallel_loop(0, lenb, unroll=2,
                                             carry=init)(k_step)

            fg = best >= IOU_THRESH
            plsc.store_scatter(ov, [oidx], best, mask=act)
            plsc.store_scatter(oi, [oidx], bcomb >> 7, mask=act)
            plsc.store_scatter(oc, [oidx],
                               jnp.where(fg, bcomb & 127, NUM_CLASSES),
                               mask=act)
            return 0

        lax.fori_loop(0, (cntb + L - 1) >> 4, pv_step, 0)
        return 0

    lax.fori_loop(0, NBINS, bin_step, 0)

    pltpu.sync_copy(ov, vals_h.at[pl.ds(base, PPW)])
    pltpu.sync_copy(oi, idxs_h.at[pl.ds(base, PPW)])
    pltpu.sync_copy(oc, cls_h.at[pl.ds(base, PPW)])


@jax.jit
def kernel(proposal_boxes, gt_boxes, gt_classes):
    pb = jnp.zeros((N_PAD, 4), jnp.float32).at[:N_PROP].set(proposal_boxes)
    gt = jnp.zeros((M_PAD, 4), jnp.float32).at[:M_GT].set(gt_boxes)
    gc = jnp.zeros((M_PAD,), jnp.int32).at[:M_GT].set(
        gt_classes.astype(jnp.int32))

    mesh = plsc.VectorSubcoreMesh(core_axis_name="c", subcore_axis_name="s")
    k = functools.partial(
        pl.kernel,
        mesh=mesh,
        compiler_params=pltpu.CompilerParams(needs_layout_passes=False),
        out_type=[
            jax.ShapeDtypeStruct((N_PAD,), jnp.float32),
            jax.ShapeDtypeStruct((N_PAD,), jnp.int32),
            jax.ShapeDtypeStruct((N_PAD,), jnp.int32),
        ],
        scratch_types=[
            pltpu.VMEM((PPW,), jnp.float32),
            pltpu.VMEM((PPW,), jnp.float32),
            pltpu.VMEM((PPW,), jnp.float32),
            pltpu.VMEM((PPW,), jnp.float32),
            pltpu.VMEM((PPW,), jnp.int32),
            pltpu.VMEM((M_PAD,), jnp.float32),
            pltpu.VMEM((M_PAD,), jnp.float32),
            pltpu.VMEM((M_PAD,), jnp.float32),
            pltpu.VMEM((M_PAD,), jnp.float32),
            pltpu.VMEM((M_PAD,), jnp.int32),
            pltpu.VMEM((M_PAD,), jnp.int32),
            pltpu.VMEM((M_PAD,), jnp.int32),
            pltpu.VMEM((NBINS * ROWLEN,), jnp.int32),
            pltpu.VMEM((NBV * L,), jnp.int32),
            pltpu.VMEM((NBINS * PPW,), jnp.int32),
            pltpu.VMEM((NBV * L,), jnp.int32),
            pltpu.VMEM((PPW,), jnp.float32),
            pltpu.VMEM((PPW,), jnp.int32),
            pltpu.VMEM((PPW,), jnp.int32),
        ],
    )(_body)

    vals, idxs, cls = k(
        pb[:, 0], pb[:, 1], pb[:, 2], pb[:, 3],
        gt[:, 0], gt[:, 1], gt[:, 2], gt[:, 3], gc,
    )
    return vals[:N_PROP], idxs[:N_PROP], cls[:N_PROP]
